# baseline (speedup 1.0000x reference)
import functools

import jax
import jax.numpy as jnp
from jax import lax
from jax.experimental import pallas as pl
from jax.experimental.pallas import tpu as pltpu
from jax.experimental.pallas import tpu_sc as plsc

N = 10000
E = 320000
F_IN = 128
H = 256
L = 3
G = 16

NW = 32
NB = 320
NPAD = NW * NB
C = 112
NCH = E // C + 3
EPAD = NCH * C

_f32 = jnp.float32


def _ln(x, g, b, eps=1e-5):
    m = jnp.mean(x, axis=-1, keepdims=True)
    v = jnp.mean((x - m) ** 2, axis=-1, keepdims=True)
    return (x - m) * lax.rsqrt(v + eps) * g + b


def _node_proj_body(x_ref, w_ref, b_ref, g_ref, bb_ref, z_ref, h_ref):
    z = jnp.dot(x_ref[...], w_ref[...], preferred_element_type=_f32) + b_ref[...]
    z_ref[...] = z
    h_ref[...] = jnp.maximum(_ln(z, g_ref[...], bb_ref[...]), 0.0)


def _edge_proj_body(a_ref, wl_ref, wh_ref, bl_ref, bh_ref, o_ref):
    ab = a_ref[...].astype(jnp.bfloat16)
    ol = jnp.dot(ab, wl_ref[...].astype(jnp.bfloat16),
                 preferred_element_type=_f32) + bl_ref[...]
    oh = jnp.dot(ab, wh_ref[...].astype(jnp.bfloat16),
                 preferred_element_type=_f32) + bh_ref[...]
    bl = lax.bitcast_convert_type(ol, jnp.int32) + jnp.int32(0x8000)
    bh = lax.bitcast_convert_type(oh, jnp.int32) + jnp.int32(0x8000)
    o_ref[...] = (bh & jnp.int32(-65536)) | lax.shift_right_logical(bl, 16)


def _mlp_body(agg_ref, h_ref, z_ref, w1_ref, b1_ref, g1_ref, bb1_ref,
              w2_ref, b2_ref, gn_ref, bn_ref, znew_ref, hnext_ref):
    out = agg_ref[...] + h_ref[...]
    o = jnp.dot(out, w1_ref[...], preferred_element_type=_f32) + b1_ref[...]
    o = jnp.maximum(_ln(o, g1_ref[...], bb1_ref[...]), 0.0)
    o = jnp.dot(o, w2_ref[...], preferred_element_type=_f32) + b2_ref[...]
    z = z_ref[...] + o
    znew_ref[...] = z
    hnext_ref[...] = jnp.maximum(_ln(z, gn_ref[...], bn_ref[...]), 0.0)


def _gs_body(oh_ref, z_ref, o_ref):
    @pl.when(pl.program_id(0) == 0)
    def _():
        o_ref[...] = jnp.zeros_like(o_ref)
    o_ref[...] += lax.dot_general(
        oh_ref[...], z_ref[...], (((0,), (0,)), ((), ())),
        preferred_element_type=_f32)


_RB = 1000


def _const_spec(shape):
    return pl.BlockSpec(shape, lambda i: tuple(0 for _ in shape))


def _node_proj(x, w, b, g, bb):
    return pl.pallas_call(
        _node_proj_body,
        grid=(N // _RB,),
        in_specs=[
            pl.BlockSpec((_RB, F_IN), lambda i: (i, 0)),
            _const_spec((F_IN, H)),
            _const_spec((1, H)),
            _const_spec((1, H)),
            _const_spec((1, H)),
        ],
        out_specs=[
            pl.BlockSpec((_RB, H), lambda i: (i, 0)),
            pl.BlockSpec((_RB, H), lambda i: (i, 0)),
        ],
        out_shape=[
            jax.ShapeDtypeStruct((N, H), _f32),
            jax.ShapeDtypeStruct((N, H), _f32),
        ],
    )(x, w, b, g, bb)


_EB = 3200


def _edge_proj(a, wl, wh, bl, bh):
    return pl.pallas_call(
        _edge_proj_body,
        grid=(E // _EB,),
        in_specs=[
            pl.BlockSpec((_EB, F_IN), lambda i: (i, 0)),
            _const_spec((F_IN, H // 2)),
            _const_spec((F_IN, H // 2)),
            _const_spec((1, H // 2)),
            _const_spec((1, H // 2)),
        ],
        out_specs=pl.BlockSpec((_EB, H // 2), lambda i: (i, 0)),
        out_shape=jax.ShapeDtypeStruct((E, H // 2), jnp.int32),
    )(a, wl, wh, bl, bh)


def _mlp(agg, h, z, w1, b1, g1, bb1, w2, b2, gn, bn):
    return pl.pallas_call(
        _mlp_body,
        grid=(N // _RB,),
        in_specs=[
            pl.BlockSpec((_RB, H), lambda i: (i, 0)),
            pl.BlockSpec((_RB, H), lambda i: (i, 0)),
            pl.BlockSpec((_RB, H), lambda i: (i, 0)),
            _const_spec((H, 2 * H)),
            _const_spec((1, 2 * H)),
            _const_spec((1, 2 * H)),
            _const_spec((1, 2 * H)),
            _const_spec((2 * H, H)),
            _const_spec((1, H)),
            _const_spec((1, H)),
            _const_spec((1, H)),
        ],
        out_specs=[
            pl.BlockSpec((_RB, H), lambda i: (i, 0)),
            pl.BlockSpec((_RB, H), lambda i: (i, 0)),
        ],
        out_shape=[
            jax.ShapeDtypeStruct((N, H), _f32),
            jax.ShapeDtypeStruct((N, H), _f32),
        ],
    )(agg, h, z, w1, b1, g1, bb1, w2, b2, gn, bn)


def _graph_sums(onehot, zcat):
    d = zcat.shape[1]
    return pl.pallas_call(
        _gs_body,
        grid=(N // _RB,),
        in_specs=[
            pl.BlockSpec((_RB, G), lambda i: (i, 0)),
            pl.BlockSpec((_RB, d), lambda i: (i, 0)),
        ],
        out_specs=pl.BlockSpec((G, d), lambda i: (0, 0)),
        out_shape=jax.ShapeDtypeStruct((G, d), _f32),
    )(onehot, zcat)


def _sload(ref, idx):
    return ref[pl.ds(idx, 16)][0]


def _sc_agg_body(h_hbm, ea_hbm, cidx_hbm, rowoff_hbm, t_hbm,
                 out_hbm, rowoff_v, t_v, cidx_a, cidx_b, hbuf_a, ebuf_a,
                 hbuf_b, ebuf_b, staging, accd, accu,
                 semh_a, seme_a, semh_b, seme_b, semf_0, semf_1):
    cc = lax.axis_index("c")
    ss = lax.axis_index("s")
    wid = ss * 2 + cc
    n0 = wid * NB

    pltpu.sync_copy(rowoff_hbm.at[pl.ds(n0, NB + 16)], rowoff_v)
    pltpu.sync_copy(t_hbm, t_v)
    ts = _sload(t_v, 0)

    zero = jnp.zeros((16,), _f32)
    for gg in range(H // 16):
        accd[pl.ds(gg * 16, 16)] = zero
        accu[pl.ds(gg * 16, 16)] = zero

    e0 = _sload(rowoff_v, 0)
    e1 = _sload(rowoff_v, NB)
    e0c = (e0 // C) * C
    nchunks = (e1 - e0c + (C - 1)) // C
    nit = (jnp.maximum(nchunks, 1) + 1) // 2

    def _issue(ci, cidx_v, hb, eb, sh, se):
        cbase = e0c + ci * C
        pltpu.sync_copy(cidx_hbm.at[cbase // C], cidx_v)
        pltpu.async_copy(h_hbm.at[cidx_v.at[0]], hb, sh)
        pltpu.async_copy(ea_hbm.at[cidx_v.at[1]], eb, se)

    def _wait(hb, eb, sh, se):
        pltpu.make_async_copy(h_hbm.at[pl.ds(0, C)], hb, sh).wait()
        pltpu.make_async_copy(ea_hbm.at[pl.ds(0, C)], eb, se).wait()

    def _finalize(n):
        k = n - n0
        slot = k % 16
        half = (k // 16) % 2
        blk = k // 16

        @pl.when(slot == 0)
        def _():
            @pl.when(jnp.logical_and(blk >= 2, half == 0))
            def _():
                pltpu.make_async_copy(staging.at[0], out_hbm.at[0], semf_0).wait()

            @pl.when(jnp.logical_and(blk >= 2, half == 1))
            def _():
                pltpu.make_async_copy(staging.at[1], out_hbm.at[0], semf_1).wait()

        for gg in range(H // 16):
            go = gg * 16
            d = accd[pl.ds(go, 16)]
            u = accu[pl.ds(go, 16)]
            staging[half, slot, pl.ds(go, 16)] = u / (d + 1e-16)
            accd[pl.ds(go, 16)] = zero
            accu[pl.ds(go, 16)] = zero

        @pl.when(slot == 15)
        def _():
            dst = out_hbm.at[n0 // 16 + blk]

            @pl.when(half == 0)
            def _():
                pltpu.async_copy(staging.at[0], dst, semf_0)

            @pl.when(half == 1)
            def _():
                pltpu.async_copy(staging.at[1], dst, semf_1)

    def _compute(ci, hb, eb, n_cur):
        cbase = e0c + ci * C
        cend = jnp.minimum(cbase + C, e1)
        lo = jnp.maximum(cbase, e0)

        def _bs_body(_, lohi):
            blo, bhi = lohi
            mid = (blo + bhi) // 2
            ge = _sload(rowoff_v, mid - n0) >= cend
            return (jnp.where(ge, blo, mid + 1), jnp.where(ge, mid, bhi))

        _, n_hi = lax.fori_loop(0, 9, _bs_body, (n_cur, n0 + NB))

        def _node_body(n, carry):
            ra = _sload(rowoff_v, n - n0)
            rb = _sload(rowoff_v, n - n0 + 1)
            a = jnp.maximum(ra, lo) - cbase
            b = jnp.minimum(rb, cend) - cbase

            @pl.when(b > a)
            def _():
                acc0 = []
                for gg in range(H // 16):
                    acc0.append(accd[pl.ds(gg * 16, 16)])
                    acc0.append(accu[pl.ds(gg * 16, 16)])

                def _edge_body(e, accs):
                    out = []
                    for kk in range(H // 32):
                        v = eb[e, pl.ds(kk * 16, 16)]
                        elo = lax.bitcast_convert_type(
                            jnp.left_shift(v, 16), _f32)
                        ehi = lax.bitcast_convert_type(
                            v & jnp.int32(-65536), _f32)
                        for half, ev in ((0, elo), (1, ehi)):
                            gg = kk * 2 + half
                            go = gg * 16
                            m = jnp.maximum(hb[e, pl.ds(go, 16)] + ev,
                                            0.0) + 1e-7
                            ex = jnp.exp(m * ts)
                            out.append(accs[2 * gg] + ex)
                            out.append(accs[2 * gg + 1] + m * ex)
                    return tuple(out)

                accs = lax.fori_loop(a, b, _edge_body, tuple(acc0))
                for gg in range(H // 16):
                    accd[pl.ds(gg * 16, 16)] = accs[2 * gg]
                    accu[pl.ds(gg * 16, 16)] = accs[2 * gg + 1]

            @pl.when(rb <= cend)
            def _():
                _finalize(n)

            return carry

        lax.fori_loop(n_cur, n_hi, _node_body, 0)
        last_done = _sload(rowoff_v, n_hi - n0) <= cend
        return jnp.maximum(jnp.where(last_done, n_hi, n_hi - 1), n_cur)

    _issue(0, cidx_a, hbuf_a, ebuf_a, semh_a, seme_a)

    def _pair_body(i, n_cur):
        c0 = 2 * i
        _issue(c0 + 1, cidx_b, hbuf_b, ebuf_b, semh_b, seme_b)
        _wait(hbuf_a, ebuf_a, semh_a, seme_a)
        n_cur = _compute(c0, hbuf_a, ebuf_a, n_cur)

        @pl.when(c0 + 2 < 2 * nit)
        def _():
            _issue(c0 + 2, cidx_a, hbuf_a, ebuf_a, semh_a, seme_a)

        _wait(hbuf_b, ebuf_b, semh_b, seme_b)
        n_cur = _compute(c0 + 1, hbuf_b, ebuf_b, n_cur)
        return n_cur

    n_fin = lax.fori_loop(0, nit, _pair_body, n0)

    def _tail_body(n, carry):
        _finalize(n)
        return carry

    lax.fori_loop(n_fin, n0 + NB, _tail_body, 0)

    pltpu.make_async_copy(staging.at[0], out_hbm.at[0], semf_0).wait()
    pltpu.make_async_copy(staging.at[1], out_hbm.at[0], semf_1).wait()


def _sc_agg(h, ea, cidx, rowoff, tpad):
    mesh = plsc.VectorSubcoreMesh(core_axis_name="c", subcore_axis_name="s")
    fn = functools.partial(
        pl.kernel,
        mesh=mesh,
        out_type=jax.ShapeDtypeStruct((NPAD // 16, 16, H), _f32),
        scratch_types=[
            pltpu.VMEM((NB + 16,), jnp.int32),
            pltpu.VMEM((16,), _f32),
            pltpu.VMEM((2, C), jnp.int32),
            pltpu.VMEM((2, C), jnp.int32),
            pltpu.VMEM((C, H), _f32),
            pltpu.VMEM((C, H // 2), jnp.int32),
            pltpu.VMEM((C, H), _f32),
            pltpu.VMEM((C, H // 2), jnp.int32),
            pltpu.VMEM((2, 16, H), _f32),
            pltpu.VMEM((H,), _f32),
            pltpu.VMEM((H,), _f32),
            pltpu.SemaphoreType.DMA,
            pltpu.SemaphoreType.DMA,
            pltpu.SemaphoreType.DMA,
            pltpu.SemaphoreType.DMA,
            pltpu.SemaphoreType.DMA,
            pltpu.SemaphoreType.DMA,
        ],
    )(_sc_agg_body)
    return fn(h, ea, cidx, rowoff, tpad)


def kernel(x, edge_index, edge_attr, batch, node_w, node_b, edge_w, edge_b,
           ln_g, ln_b, t, mlp_w1, mlp_b1, mlp_ln_g, mlp_ln_b, mlp_w2, mlp_b2):
    src, dst = edge_index[0], edge_index[1]

    perm = jnp.argsort(dst).astype(jnp.int32)
    dst_s = dst[perm]
    sidx = src[perm]
    counts = jnp.zeros((NPAD + 16,), jnp.int32).at[dst].add(1)
    rowoff = (jnp.cumsum(counts) - counts).astype(jnp.int32)
    sidx_p = jnp.zeros((EPAD,), jnp.int32).at[:E].set(sidx).reshape(NCH, 1, C)
    eidx_p = jnp.zeros((EPAD,), jnp.int32).at[:E].set(perm).reshape(NCH, 1, C)
    cidx = jnp.concatenate([sidx_p, eidx_p], axis=1)

    r2 = lambda v: v.reshape(1, -1)

    widx = jnp.arange(H // 2)
    lo_idx = (widx // 16) * 32 + widx % 16
    hi_idx = lo_idx + 16

    z, h = _node_proj(x, node_w, r2(node_b), r2(ln_g[0]), r2(ln_b[0]))
    ea = _edge_proj(edge_attr, edge_w[:, lo_idx], edge_w[:, hi_idx],
                    r2(edge_b[lo_idx]), r2(edge_b[hi_idx]))

    zs = []
    for i in range(L):
        tpad = jnp.zeros((16,), _f32).at[0].set(t[i])
        agg = _sc_agg(h, ea, cidx, rowoff, tpad).reshape(NPAD, H)[:N]
        j = i + 1 if i + 1 < L else i
        z, h = _mlp(agg, h, z, mlp_w1[i], r2(mlp_b1[i]), r2(mlp_ln_g[i]),
                    r2(mlp_ln_b[i]), mlp_w2[i], r2(mlp_b2[i]),
                    r2(ln_g[j]), r2(ln_b[j]))
        zs.append(z)

    zcat = jnp.concatenate(zs, axis=1)
    onehot = (batch[:, None] == jnp.arange(G, dtype=batch.dtype)[None, :]
              ).astype(_f32)
    gs = _graph_sums(onehot, zcat)
    return zcat, gs


# speedup vs baseline: 1.1277x; 1.1277x over previous
import functools

import jax
import jax.numpy as jnp
from jax import lax
from jax.experimental import pallas as pl
from jax.experimental.pallas import tpu as pltpu
from jax.experimental.pallas import tpu_sc as plsc

N = 10000
E = 320000
F_IN = 128
H = 256
L = 3
G = 16

NW = 32
NB = 320
NPAD = NW * NB
C = 112
NCH = E // C + 3
EPAD = NCH * C

_f32 = jnp.float32


def _ln(x, g, b, eps=1e-5):
    m = jnp.mean(x, axis=-1, keepdims=True)
    v = jnp.mean((x - m) ** 2, axis=-1, keepdims=True)
    return (x - m) * lax.rsqrt(v + eps) * g + b


def _node_proj_body(x_ref, w_ref, b_ref, g_ref, bb_ref, z_ref, h_ref):
    z = jnp.dot(x_ref[...], w_ref[...], preferred_element_type=_f32) + b_ref[...]
    z_ref[...] = z
    h_ref[...] = jnp.maximum(_ln(z, g_ref[...], bb_ref[...]), 0.0)


def _edge_proj_body(a_ref, w_ref, b_ref, o_ref):
    o_ref[...] = jnp.dot(a_ref[...].astype(jnp.bfloat16),
                         w_ref[...].astype(jnp.bfloat16),
                         preferred_element_type=_f32) + b_ref[...]


def _mlp_body(agg_ref, h_ref, z_ref, w1_ref, b1_ref, g1_ref, bb1_ref,
              w2_ref, b2_ref, gn_ref, bn_ref, znew_ref, hnext_ref):
    out = agg_ref[...] + h_ref[...]
    o = jnp.dot(out, w1_ref[...], preferred_element_type=_f32) + b1_ref[...]
    o = jnp.maximum(_ln(o, g1_ref[...], bb1_ref[...]), 0.0)
    o = jnp.dot(o, w2_ref[...], preferred_element_type=_f32) + b2_ref[...]
    z = z_ref[...] + o
    znew_ref[...] = z
    hnext_ref[...] = jnp.maximum(_ln(z, gn_ref[...], bn_ref[...]), 0.0)


def _gs_body(oh_ref, z_ref, o_ref):
    @pl.when(pl.program_id(0) == 0)
    def _():
        o_ref[...] = jnp.zeros_like(o_ref)
    o_ref[...] += lax.dot_general(
        oh_ref[...], z_ref[...], (((0,), (0,)), ((), ())),
        preferred_element_type=_f32)


_RB = 1000


def _const_spec(shape):
    return pl.BlockSpec(shape, lambda i: tuple(0 for _ in shape))


def _node_proj(x, w, b, g, bb):
    return pl.pallas_call(
        _node_proj_body,
        grid=(N // _RB,),
        in_specs=[
            pl.BlockSpec((_RB, F_IN), lambda i: (i, 0)),
            _const_spec((F_IN, H)),
            _const_spec((1, H)),
            _const_spec((1, H)),
            _const_spec((1, H)),
        ],
        out_specs=[
            pl.BlockSpec((_RB, H), lambda i: (i, 0)),
            pl.BlockSpec((_RB, H), lambda i: (i, 0)),
        ],
        out_shape=[
            jax.ShapeDtypeStruct((N, H), _f32),
            jax.ShapeDtypeStruct((N, H), _f32),
        ],
    )(x, w, b, g, bb)


_EB = 3200


def _edge_proj(a, w, b):
    return pl.pallas_call(
        _edge_proj_body,
        grid=(E // _EB,),
        in_specs=[
            pl.BlockSpec((_EB, F_IN), lambda i: (i, 0)),
            _const_spec((F_IN, H)),
            _const_spec((1, H)),
        ],
        out_specs=pl.BlockSpec((_EB, H), lambda i: (i, 0)),
        out_shape=jax.ShapeDtypeStruct((E, H), _f32),
    )(a, w, b)


def _mlp(agg, h, z, w1, b1, g1, bb1, w2, b2, gn, bn):
    return pl.pallas_call(
        _mlp_body,
        grid=(N // _RB,),
        in_specs=[
            pl.BlockSpec((_RB, H), lambda i: (i, 0)),
            pl.BlockSpec((_RB, H), lambda i: (i, 0)),
            pl.BlockSpec((_RB, H), lambda i: (i, 0)),
            _const_spec((H, 2 * H)),
            _const_spec((1, 2 * H)),
            _const_spec((1, 2 * H)),
            _const_spec((1, 2 * H)),
            _const_spec((2 * H, H)),
            _const_spec((1, H)),
            _const_spec((1, H)),
            _const_spec((1, H)),
        ],
        out_specs=[
            pl.BlockSpec((_RB, H), lambda i: (i, 0)),
            pl.BlockSpec((_RB, H), lambda i: (i, 0)),
        ],
        out_shape=[
            jax.ShapeDtypeStruct((N, H), _f32),
            jax.ShapeDtypeStruct((N, H), _f32),
        ],
    )(agg, h, z, w1, b1, g1, bb1, w2, b2, gn, bn)


def _graph_sums(onehot, zcat):
    d = zcat.shape[1]
    return pl.pallas_call(
        _gs_body,
        grid=(N // _RB,),
        in_specs=[
            pl.BlockSpec((_RB, G), lambda i: (i, 0)),
            pl.BlockSpec((_RB, d), lambda i: (i, 0)),
        ],
        out_specs=pl.BlockSpec((G, d), lambda i: (0, 0)),
        out_shape=jax.ShapeDtypeStruct((G, d), _f32),
    )(onehot, zcat)


def _sload(ref, idx):
    return ref[pl.ds(idx, 16)][0]


def _sc_agg_body(h_hbm, ea_hbm, cidx_hbm, rowoff_hbm, t_hbm,
                 out_hbm, rowoff_v, t_v, cidx_a, cidx_b, hbuf_a, ebuf_a,
                 hbuf_b, ebuf_b, staging, accd, accu,
                 semh_a, seme_a, semh_b, seme_b, semf_0, semf_1):
    cc = lax.axis_index("c")
    ss = lax.axis_index("s")
    wid = ss * 2 + cc
    n0 = wid * NB

    pltpu.sync_copy(rowoff_hbm.at[pl.ds(n0, NB + 16)], rowoff_v)
    pltpu.sync_copy(t_hbm, t_v)
    ts = _sload(t_v, 0)

    zero = jnp.zeros((16,), _f32)
    for gg in range(H // 16):
        accd[pl.ds(gg * 16, 16)] = zero
        accu[pl.ds(gg * 16, 16)] = zero

    e0 = _sload(rowoff_v, 0)
    e1 = _sload(rowoff_v, NB)
    e0c = (e0 // C) * C
    nchunks = (e1 - e0c + (C - 1)) // C
    nit = (jnp.maximum(nchunks, 1) + 1) // 2

    def _issue(ci, cidx_v, hb, eb, sh, se):
        cbase = e0c + ci * C
        pltpu.sync_copy(cidx_hbm.at[cbase // C], cidx_v)
        pltpu.async_copy(h_hbm.at[cidx_v.at[0]], hb, sh)
        pltpu.async_copy(ea_hbm.at[cidx_v.at[1]], eb, se)

    def _wait(hb, eb, sh, se):
        pltpu.make_async_copy(h_hbm.at[pl.ds(0, C)], hb, sh).wait()
        pltpu.make_async_copy(ea_hbm.at[pl.ds(0, C)], eb, se).wait()

    def _finalize(n):
        k = n - n0
        slot = k % 16
        half = (k // 16) % 2
        blk = k // 16

        @pl.when(slot == 0)
        def _():
            @pl.when(jnp.logical_and(blk >= 2, half == 0))
            def _():
                pltpu.make_async_copy(staging.at[0], out_hbm.at[0], semf_0).wait()

            @pl.when(jnp.logical_and(blk >= 2, half == 1))
            def _():
                pltpu.make_async_copy(staging.at[1], out_hbm.at[0], semf_1).wait()

        for gg in range(H // 16):
            go = gg * 16
            d = accd[pl.ds(go, 16)]
            u = accu[pl.ds(go, 16)]
            staging[half, slot, pl.ds(go, 16)] = u / (d + 1e-16)
            accd[pl.ds(go, 16)] = zero
            accu[pl.ds(go, 16)] = zero

        @pl.when(slot == 15)
        def _():
            dst = out_hbm.at[n0 // 16 + blk]

            @pl.when(half == 0)
            def _():
                pltpu.async_copy(staging.at[0], dst, semf_0)

            @pl.when(half == 1)
            def _():
                pltpu.async_copy(staging.at[1], dst, semf_1)

    def _compute(ci, hb, eb, n_cur):
        cbase = e0c + ci * C
        cend = jnp.minimum(cbase + C, e1)
        lo = jnp.maximum(cbase, e0)

        def _bs_body(_, lohi):
            blo, bhi = lohi
            mid = (blo + bhi) // 2
            ge = _sload(rowoff_v, mid - n0) >= cend
            return (jnp.where(ge, blo, mid + 1), jnp.where(ge, mid, bhi))

        _, n_hi = lax.fori_loop(0, 9, _bs_body, (n_cur, n0 + NB))

        def _node_body(n, carry):
            ra = _sload(rowoff_v, n - n0)
            rb = _sload(rowoff_v, n - n0 + 1)
            a = jnp.maximum(ra, lo) - cbase
            b = jnp.minimum(rb, cend) - cbase

            @pl.when(b > a)
            def _():
                acc0 = []
                for gg in range(H // 16):
                    acc0.append(accd[pl.ds(gg * 16, 16)])
                    acc0.append(accu[pl.ds(gg * 16, 16)])

                def _edge_body(e, accs):
                    out = []
                    for gg in range(H // 16):
                        go = gg * 16
                        m = jnp.maximum(
                            hb[e, pl.ds(go, 16)] + eb[e, pl.ds(go, 16)],
                            0.0) + 1e-7
                        ex = jnp.exp(m * ts)
                        out.append(accs[2 * gg] + ex)
                        out.append(accs[2 * gg + 1] + m * ex)
                    return tuple(out)

                accs = lax.fori_loop(a, b, _edge_body, tuple(acc0))
                for gg in range(H // 16):
                    accd[pl.ds(gg * 16, 16)] = accs[2 * gg]
                    accu[pl.ds(gg * 16, 16)] = accs[2 * gg + 1]

            @pl.when(rb <= cend)
            def _():
                _finalize(n)

            return carry

        lax.fori_loop(n_cur, n_hi, _node_body, 0)
        last_done = _sload(rowoff_v, n_hi - n0) <= cend
        return jnp.maximum(jnp.where(last_done, n_hi, n_hi - 1), n_cur)

    _issue(0, cidx_a, hbuf_a, ebuf_a, semh_a, seme_a)

    def _pair_body(i, n_cur):
        c0 = 2 * i
        _issue(c0 + 1, cidx_b, hbuf_b, ebuf_b, semh_b, seme_b)
        _wait(hbuf_a, ebuf_a, semh_a, seme_a)
        n_cur = _compute(c0, hbuf_a, ebuf_a, n_cur)

        @pl.when(c0 + 2 < 2 * nit)
        def _():
            _issue(c0 + 2, cidx_a, hbuf_a, ebuf_a, semh_a, seme_a)

        _wait(hbuf_b, ebuf_b, semh_b, seme_b)
        n_cur = _compute(c0 + 1, hbuf_b, ebuf_b, n_cur)
        return n_cur

    n_fin = lax.fori_loop(0, nit, _pair_body, n0)

    def _tail_body(n, carry):
        _finalize(n)
        return carry

    lax.fori_loop(n_fin, n0 + NB, _tail_body, 0)

    pltpu.make_async_copy(staging.at[0], out_hbm.at[0], semf_0).wait()
    pltpu.make_async_copy(staging.at[1], out_hbm.at[0], semf_1).wait()


def _sc_agg(h, ea, cidx, rowoff, tpad):
    mesh = plsc.VectorSubcoreMesh(core_axis_name="c", subcore_axis_name="s")
    fn = functools.partial(
        pl.kernel,
        mesh=mesh,
        out_type=jax.ShapeDtypeStruct((NPAD // 16, 16, H), _f32),
        scratch_types=[
            pltpu.VMEM((NB + 16,), jnp.int32),
            pltpu.VMEM((16,), _f32),
            pltpu.VMEM((2, C), jnp.int32),
            pltpu.VMEM((2, C), jnp.int32),
            pltpu.VMEM((C, H), _f32),
            pltpu.VMEM((C, H), _f32),
            pltpu.VMEM((C, H), _f32),
            pltpu.VMEM((C, H), _f32),
            pltpu.VMEM((2, 16, H), _f32),
            pltpu.VMEM((H,), _f32),
            pltpu.VMEM((H,), _f32),
            pltpu.SemaphoreType.DMA,
            pltpu.SemaphoreType.DMA,
            pltpu.SemaphoreType.DMA,
            pltpu.SemaphoreType.DMA,
            pltpu.SemaphoreType.DMA,
            pltpu.SemaphoreType.DMA,
        ],
    )(_sc_agg_body)
    return fn(h, ea, cidx, rowoff, tpad)


def kernel(x, edge_index, edge_attr, batch, node_w, node_b, edge_w, edge_b,
           ln_g, ln_b, t, mlp_w1, mlp_b1, mlp_ln_g, mlp_ln_b, mlp_w2, mlp_b2):
    src, dst = edge_index[0], edge_index[1]

    perm = jnp.argsort(dst).astype(jnp.int32)
    dst_s = dst[perm]
    sidx = src[perm]
    counts = jnp.zeros((NPAD + 16,), jnp.int32).at[dst].add(1)
    rowoff = (jnp.cumsum(counts) - counts).astype(jnp.int32)
    sidx_p = jnp.zeros((EPAD,), jnp.int32).at[:E].set(sidx).reshape(NCH, 1, C)
    eidx_p = jnp.zeros((EPAD,), jnp.int32).at[:E].set(perm).reshape(NCH, 1, C)
    cidx = jnp.concatenate([sidx_p, eidx_p], axis=1)

    r2 = lambda v: v.reshape(1, -1)

    z, h = _node_proj(x, node_w, r2(node_b), r2(ln_g[0]), r2(ln_b[0]))
    ea = _edge_proj(edge_attr, edge_w, r2(edge_b))

    zs = []
    for i in range(L):
        tpad = jnp.zeros((16,), _f32).at[0].set(t[i])
        agg = _sc_agg(h, ea, cidx, rowoff, tpad).reshape(NPAD, H)[:N]
        j = i + 1 if i + 1 < L else i
        z, h = _mlp(agg, h, z, mlp_w1[i], r2(mlp_b1[i]), r2(mlp_ln_g[i]),
                    r2(mlp_ln_b[i]), mlp_w2[i], r2(mlp_b2[i]),
                    r2(ln_g[j]), r2(ln_b[j]))
        zs.append(z)

    zcat = jnp.concatenate(zs, axis=1)
    onehot = (batch[:, None] == jnp.arange(G, dtype=batch.dtype)[None, :]
              ).astype(_f32)
    gs = _graph_sums(onehot, zcat)
    return zcat, gs
